# baseline (speedup 1.0000x reference)
import functools

import jax
import jax.numpy as jnp
from jax import lax
from jax.experimental import pallas as pl
from jax.experimental.pallas import tpu as pltpu
from jax.experimental.pallas import tpu_sc as plsc

F32 = jnp.float32
BF16 = jnp.bfloat16
_INTERPRET = False


def _pc(*a, **k):
    return pl.pallas_call(*a, interpret=_INTERPRET, **k)


def _ln(v, g, b):
    mu = jnp.mean(v, axis=-1, keepdims=True)
    var = jnp.mean((v - mu) * (v - mu), axis=-1, keepdims=True)
    return (v - mu) * jax.lax.rsqrt(var + 1e-5) * g + b


def _dot(a, b, prec=None):
    return jax.lax.dot_general(a, b, (((1,), (0,)), ((), ())),
                               preferred_element_type=F32, precision=prec)


_HI = jax.lax.Precision.HIGHEST


def _emb_body(x_ref, w_ref, b_ref, h_ref):
    h_ref[...] = _dot(x_ref[...], w_ref[...], _HI) + b_ref[...]


def _emb(x, w, b):
    n = x.shape[0]
    return _pc(
        _emb_body,
        out_shape=jax.ShapeDtypeStruct((n, w.shape[1]), F32),
    )(x, w, b.reshape(1, -1))


def _tables_body(h_ref, wa_ref, wb_ref, b1_ref, pa_ref, pb_ref):
    h = h_ref[...]
    pa_ref[...] = (_dot(h, wa_ref[...], _HI) + b1_ref[...]).astype(BF16)
    pb_ref[...] = _dot(h, wb_ref[...], _HI).astype(BF16)


def _tables(h, w1, b1, bn):
    n, d = h.shape
    wa = w1[:d]
    wb = w1[d:2 * d]
    grid = n // bn
    return _pc(
        _tables_body,
        grid=(grid,),
        in_specs=[
            pl.BlockSpec((bn, d), lambda i: (i, 0)),
            pl.BlockSpec((d, d), lambda i: (0, 0)),
            pl.BlockSpec((d, d), lambda i: (0, 0)),
            pl.BlockSpec((1, d), lambda i: (0, 0)),
        ],
        out_specs=[
            pl.BlockSpec((bn, d), lambda i: (i, 0)),
            pl.BlockSpec((bn, d), lambda i: (i, 0)),
        ],
        out_shape=[
            jax.ShapeDtypeStruct((n, d), BF16),
            jax.ShapeDtypeStruct((n, d), BF16),
        ],
    )(h, wa, wb, b1.reshape(1, -1))


def _edge_body(ga_ref, gb_ref, gpd_ref, gps_ref, w1d_ref,
               g1_ref, be1_ref, w2_ref, b2_ref, g2_ref, be2_ref,
               pw1_ref, pb1_ref, pg_ref, pbe_ref, pw2_ref, pb2_ref,
               m_ref, aux_ref):
    pd = gpd_ref[...][:, :3] - gps_ref[...][:, :3]
    d2 = jnp.sum(pd * pd, axis=-1, keepdims=True)
    dist = jnp.sqrt(d2 + 1e-12)
    m1 = (ga_ref[...].astype(F32) + gb_ref[...].astype(F32)
          + dist * w1d_ref[...])
    m1 = jax.nn.relu(_ln(m1, g1_ref[...], be1_ref[...]))
    m = _dot(m1, w2_ref[...]) + b2_ref[...]
    m = jax.nn.relu(_ln(m, g2_ref[...], be2_ref[...]))
    w = _dot(m, pw1_ref[...]) + pb1_ref[...]
    w = jax.nn.relu(_ln(w, pg_ref[...], pbe_ref[...]))
    s = jnp.sum(w * pw2_ref[...], axis=-1, keepdims=True) + pb2_ref[0, 0]
    pdw = pd * s
    m_ref[...] = m
    be = pd.shape[0]
    aux = jnp.concatenate(
        [pdw, jnp.ones((be, 1), F32), jnp.zeros((be, 12), F32)], axis=-1)
    aux_ref[...] = aux


def _edge_mlp(ga, gb, gpd, gps, p, l, be):
    e, d = ga.shape
    grid = e // be
    row = lambda a: a.reshape(1, -1)
    w1d = p["msg_W1"][l][2 * d].reshape(1, d)
    return _pc(
        _edge_body,
        grid=(grid,),
        in_specs=[
            pl.BlockSpec((be, d), lambda i: (i, 0)),
            pl.BlockSpec((be, d), lambda i: (i, 0)),
            pl.BlockSpec((be, 16), lambda i: (i, 0)),
            pl.BlockSpec((be, 16), lambda i: (i, 0)),
            pl.BlockSpec((1, d), lambda i: (0, 0)),
            pl.BlockSpec((1, d), lambda i: (0, 0)),
            pl.BlockSpec((1, d), lambda i: (0, 0)),
            pl.BlockSpec((d, d), lambda i: (0, 0)),
            pl.BlockSpec((1, d), lambda i: (0, 0)),
            pl.BlockSpec((1, d), lambda i: (0, 0)),
            pl.BlockSpec((1, d), lambda i: (0, 0)),
            pl.BlockSpec((d, d), lambda i: (0, 0)),
            pl.BlockSpec((1, d), lambda i: (0, 0)),
            pl.BlockSpec((1, d), lambda i: (0, 0)),
            pl.BlockSpec((1, d), lambda i: (0, 0)),
            pl.BlockSpec((1, d), lambda i: (0, 0)),
            pl.BlockSpec((1, 1), lambda i: (0, 0)),
        ],
        out_specs=[
            pl.BlockSpec((be, d), lambda i: (i, 0)),
            pl.BlockSpec((be, 16), lambda i: (i, 0)),
        ],
        out_shape=[
            jax.ShapeDtypeStruct((e, d), F32),
            jax.ShapeDtypeStruct((e, 16), F32),
        ],
    )(ga, gb, gpd, gps, w1d,
      row(p["msg_g1"][l]), row(p["msg_be1"][l]),
      p["msg_W2"][l], row(p["msg_b2"][l]),
      row(p["msg_g2"][l]), row(p["msg_be2"][l]),
      p["pos_W1"][l], row(p["pos_b1"][l]),
      row(p["pos_g"][l]), row(p["pos_be"][l]),
      p["pos_W2"][l].reshape(1, d), p["pos_b2"][l].reshape(1, 1))


def _node_body(h_ref, posp_ref, pm_ref, paux_ref,
               u1a_ref, u1b_ref, ub1_ref, ug1_ref, ube1_ref,
               u2_ref, ub2_ref, ug2_ref, ube2_ref,
               hn_ref, posn_ref):
    h = h_ref[...]
    msg = pm_ref[0] + pm_ref[1]
    aux = paux_ref[0] + paux_ref[1]
    cnt = jnp.maximum(aux[:, 3:4], 1.0)
    pos_aggr = aux[:, :3] / cnt
    u = _dot(h, u1a_ref[...], _HI) + _dot(msg, u1b_ref[...], _HI) + ub1_ref[...]
    u = jax.nn.relu(_ln(u, ug1_ref[...], ube1_ref[...]))
    u = _dot(u, u2_ref[...], _HI) + ub2_ref[...]
    u = jax.nn.relu(_ln(u, ug2_ref[...], ube2_ref[...]))
    hn_ref[...] = h + u
    be = h.shape[0]
    posn_ref[...] = posp_ref[...] + jnp.concatenate(
        [pos_aggr, jnp.zeros((be, 13), F32)], axis=-1)


def _node_update(h, posp, pm, paux, p, l, bn):
    n, d = h.shape
    grid = n // bn
    row = lambda a: a.reshape(1, -1)
    u1 = p["upd_W1"][l]
    return _pc(
        _node_body,
        grid=(grid,),
        in_specs=[
            pl.BlockSpec((bn, d), lambda i: (i, 0)),
            pl.BlockSpec((bn, 16), lambda i: (i, 0)),
            pl.BlockSpec((2, bn, d), lambda i: (0, i, 0)),
            pl.BlockSpec((2, bn, 16), lambda i: (0, i, 0)),
            pl.BlockSpec((d, d), lambda i: (0, 0)),
            pl.BlockSpec((d, d), lambda i: (0, 0)),
            pl.BlockSpec((1, d), lambda i: (0, 0)),
            pl.BlockSpec((1, d), lambda i: (0, 0)),
            pl.BlockSpec((1, d), lambda i: (0, 0)),
            pl.BlockSpec((d, d), lambda i: (0, 0)),
            pl.BlockSpec((1, d), lambda i: (0, 0)),
            pl.BlockSpec((1, d), lambda i: (0, 0)),
            pl.BlockSpec((1, d), lambda i: (0, 0)),
        ],
        out_specs=[
            pl.BlockSpec((bn, d), lambda i: (i, 0)),
            pl.BlockSpec((bn, 16), lambda i: (i, 0)),
        ],
        out_shape=[
            jax.ShapeDtypeStruct((n, d), F32),
            jax.ShapeDtypeStruct((n, 16), F32),
        ],
    )(h, posp, pm, paux,
      u1[:d], u1[d:2 * d], row(p["upd_b1"][l]),
      row(p["upd_g1"][l]), row(p["upd_be1"][l]),
      p["upd_W2"][l], row(p["upd_b2"][l]),
      row(p["upd_g2"][l]), row(p["upd_be2"][l]))


def _pred_body(h_ref, b_ref, w1_ref, b1_ref, w2_ref, b2_ref, o_ref):
    n, _ = h_ref.shape
    ng = o_ref.shape[0]
    seg = jax.lax.broadcasted_iota(jnp.int32, (n, ng), 1)
    onehot = jnp.where(b_ref[...] == seg, 1.0, 0.0).astype(F32)
    g = jax.lax.dot_general(onehot, h_ref[...], (((0,), (0,)), ((), ())),
                            preferred_element_type=F32, precision=_HI)
    u = jax.nn.relu(_dot(g, w1_ref[...], _HI) + b1_ref[...])
    o_ref[...] = _dot(u, w2_ref[...], _HI) + b2_ref[...]


def _pred(h, batch, p, ng):
    n, d = h.shape
    out_f = p["pred_W2"].shape[1]
    return _pc(
        _pred_body,
        out_shape=jax.ShapeDtypeStruct((ng, out_f), F32),
    )(h, batch.reshape(n, 1), p["pred_W1"], p["pred_b1"].reshape(1, -1),
      p["pred_W2"], p["pred_b2"].reshape(1, -1))


_NW = 32
_CHUNK = 80


def _sc_gather(pa, pb, posp, src2, dst2):
    n, d = pa.shape
    e = src2.shape[0] * src2.shape[1]
    per_w = e // _NW
    nch = per_w // _CHUNK
    assert per_w * _NW == e and nch * _CHUNK == per_w and nch % 2 == 1
    mesh = plsc.VectorSubcoreMesh(core_axis_name="c", subcore_axis_name="s")

    @functools.partial(
        pl.kernel, mesh=mesh, interpret=_INTERPRET,
        compiler_params=pltpu.CompilerParams(use_tc_tiling_on_sc=False),
        out_type=[jax.ShapeDtypeStruct((e, d), BF16),
                  jax.ShapeDtypeStruct((e, d), BF16),
                  jax.ShapeDtypeStruct((e, 16), F32),
                  jax.ShapeDtypeStruct((e, 16), F32)],
        scratch_types=[pltpu.VMEM((nch, _CHUNK), jnp.int32),
                       pltpu.VMEM((nch, _CHUNK), jnp.int32)]
                      + [pltpu.VMEM((_CHUNK, d), BF16),
                         pltpu.VMEM((_CHUNK, d), BF16),
                         pltpu.VMEM((_CHUNK, 16), F32),
                         pltpu.VMEM((_CHUNK, 16), F32)] * 2
                      + [pltpu.SemaphoreType.DMA] * 4,
    )
    def k(pa_h, pb_h, posp_h, src_h, dst_h, ga_h, gb_h, gpd_h, gps_h,
          idxd_v, idxs_v, ra0, rb0, rpd0, rps0, ra1, rb1, rpd1, rps1,
          gsem0, gsem1, wsem0, wsem1):
        wid = lax.axis_index("s") * 2 + lax.axis_index("c")
        base = wid * per_w
        pltpu.sync_copy(dst_h.at[pl.ds(wid * nch, nch)], idxd_v)
        pltpu.sync_copy(src_h.at[pl.ds(wid * nch, nch)], idxs_v)
        bufs = ((ra0, rb0, rpd0, rps0, gsem0, wsem0),
                (ra1, rb1, rpd1, rps1, gsem1, wsem1))

        def g_descs(ci, bs):
            return (pltpu.make_async_copy(pa_h.at[idxd_v.at[ci]], bs[0], bs[4]),
                    pltpu.make_async_copy(pb_h.at[idxs_v.at[ci]], bs[1], bs[4]),
                    pltpu.make_async_copy(posp_h.at[idxd_v.at[ci]], bs[2], bs[4]),
                    pltpu.make_async_copy(posp_h.at[idxs_v.at[ci]], bs[3], bs[4]))

        def w_descs(ci, bs):
            off = base + ci * _CHUNK
            return (pltpu.make_async_copy(bs[0], ga_h.at[pl.ds(off, _CHUNK)], bs[5]),
                    pltpu.make_async_copy(bs[1], gb_h.at[pl.ds(off, _CHUNK)], bs[5]),
                    pltpu.make_async_copy(bs[2], gpd_h.at[pl.ds(off, _CHUNK)], bs[5]),
                    pltpu.make_async_copy(bs[3], gps_h.at[pl.ds(off, _CHUNK)], bs[5]))

        def start(ds):
            for dd in ds:
                dd.start()

        def wait(ds):
            for dd in ds:
                dd.wait()

        start(g_descs(0, bufs[0]))

        def body(i2, carry):
            i = i2 * 2
            a, b = bufs

            @pl.when(i > 0)
            def _():
                wait(w_descs(i - 1, b))

            start(g_descs(i + 1, b))
            wait(g_descs(i, a))
            start(w_descs(i, a))
            wait(g_descs(i + 1, b))
            wait(w_descs(i, a))

            @pl.when(i + 2 < nch)
            def _():
                start(g_descs(i + 2, a))

            start(w_descs(i + 1, b))
            return carry

        lax.fori_loop(0, (nch - 1) // 2, body, 0)
        last = nch - 1
        wait(w_descs(last - 1, bufs[1]))
        wait(g_descs(last, bufs[0]))
        start(w_descs(last, bufs[0]))
        wait(w_descs(last, bufs[0]))

    return k(pa, pb, posp, src2, dst2)


def _sc_scatter(m, aux, dst2, n):
    e, d = m.shape
    per_w = e // _NW
    nch = per_w // _CHUNK
    assert per_w * _NW == e and nch * _CHUNK == per_w and nch % 2 == 1
    rows_s = n // 16
    assert rows_s * 16 == n
    mesh = plsc.VectorSubcoreMesh(core_axis_name="c", subcore_axis_name="s")
    zm = jnp.zeros((rows_s, d), F32)
    za = jnp.zeros((rows_s, 16), F32)

    @functools.partial(
        pl.kernel, mesh=mesh, interpret=_INTERPRET,
        compiler_params=pltpu.CompilerParams(use_tc_tiling_on_sc=False),
        out_type=[jax.ShapeDtypeStruct((2, n, d), F32),
                  jax.ShapeDtypeStruct((2, n, 16), F32)],
        scratch_types=[pltpu.VMEM((nch, _CHUNK), jnp.int32)]
                      + [pltpu.VMEM((_CHUNK, d), F32),
                         pltpu.VMEM((_CHUNK, 16), F32)] * 2
                      + [pltpu.VMEM_SHARED((n, d), F32),
                         pltpu.VMEM_SHARED((n, 16), F32)]
                      + [pltpu.SemaphoreType.DMA] * 4,
    )
    def k(m_h, aux_h, dst_h, zm_h, za_h, pm_h, paux_h,
          idx_v, rm0, rx0, rm1, rx1, accm_s, acca_s,
          rsem0, rsem1, ssem0, ssem1):
        cid = lax.axis_index("c")
        sid = lax.axis_index("s")
        wid = sid * 2 + cid
        base = wid * per_w
        pltpu.sync_copy(zm_h, accm_s.at[pl.ds(sid * rows_s, rows_s)])
        pltpu.sync_copy(za_h, acca_s.at[pl.ds(sid * rows_s, rows_s)])
        pltpu.sync_copy(dst_h.at[pl.ds(wid * nch, nch)], idx_v)
        plsc.subcore_barrier()
        bufs = ((rm0, rx0, rsem0, ssem0), (rm1, rx1, rsem1, ssem1))

        def r_descs(ci, bs):
            off = base + ci * _CHUNK
            return (pltpu.make_async_copy(m_h.at[pl.ds(off, _CHUNK)], bs[0], bs[2]),
                    pltpu.make_async_copy(aux_h.at[pl.ds(off, _CHUNK)], bs[1], bs[2]))

        def s_descs(ci, bs):
            return (pltpu.make_async_copy(bs[0], accm_s.at[idx_v.at[ci]], bs[3]),
                    pltpu.make_async_copy(bs[1], acca_s.at[idx_v.at[ci]], bs[3]))

        def start(ds, add=False):
            for dd in ds:
                dd.start(add=add)

        def wait(ds):
            for dd in ds:
                dd.wait()

        start(r_descs(0, bufs[0]))

        def body(i2, carry):
            i = i2 * 2
            a, b = bufs

            @pl.when(i > 0)
            def _():
                wait(s_descs(i - 1, b))

            start(r_descs(i + 1, b))
            wait(r_descs(i, a))
            start(s_descs(i, a), add=True)
            wait(r_descs(i + 1, b))
            wait(s_descs(i, a))

            @pl.when(i + 2 < nch)
            def _():
                start(r_descs(i + 2, a))

            start(s_descs(i + 1, b), add=True)
            return carry

        lax.fori_loop(0, (nch - 1) // 2, body, 0)
        last = nch - 1
        wait(s_descs(last - 1, bufs[1]))
        wait(r_descs(last, bufs[0]))
        start(s_descs(last, bufs[0]), add=True)
        wait(s_descs(last, bufs[0]))
        plsc.subcore_barrier()
        pltpu.sync_copy(accm_s.at[pl.ds(sid * rows_s, rows_s)],
                        pm_h.at[cid, pl.ds(sid * rows_s, rows_s)])
        pltpu.sync_copy(acca_s.at[pl.ds(sid * rows_s, rows_s)],
                        paux_h.at[cid, pl.ds(sid * rows_s, rows_s)])

    return k(m, aux, dst2, zm, za)


def _gather(pa, pb, posp, src, dst):
    ga = jnp.take(pa, dst, axis=0)
    gb = jnp.take(pb, src, axis=0)
    gpd = jnp.take(posp, dst, axis=0)
    gps = jnp.take(posp, src, axis=0)
    return ga, gb, gpd, gps


def _scatter(m, aux, dst, n):
    pm = jax.ops.segment_sum(m, dst, num_segments=n)
    paux = jax.ops.segment_sum(aux, dst, num_segments=n)
    z = jnp.zeros_like(pm)
    za = jnp.zeros_like(paux)
    return jnp.stack([pm, z]), jnp.stack([paux, za])


def kernel(x, pos, params, edge_index, batch):
    p = params
    n = x.shape[0]
    e = edge_index.shape[1]
    depth = p["msg_W1"].shape[0]
    src2 = edge_index[0].reshape(e // _CHUNK, _CHUNK)
    dst2 = edge_index[1].reshape(e // _CHUNK, _CHUNK)
    bn = 2000 if n % 2000 == 0 else n
    be = 2000 if e % 2000 == 0 else e

    posp = jnp.pad(pos, ((0, 0), (0, 13)))
    h = _emb(x, p["emb_W"], p["emb_b"])
    for l in range(depth):
        pa, pb = _tables(h, p["msg_W1"][l], p["msg_b1"][l], bn)
        ga, gb, gpd, gps = _sc_gather(pa, pb, posp, src2, dst2)
        m, aux = _edge_mlp(ga, gb, gpd, gps, p, l, be)
        pm, paux = _sc_scatter(m, aux, dst2, n)
        h, posp = _node_update(h, posp, pm, paux, p, l, bn)
    return _pred(h, batch, p, 16)


# speedup vs baseline: 1.5206x; 1.5206x over previous
import functools

import jax
import jax.numpy as jnp
from jax import lax
from jax.experimental import pallas as pl
from jax.experimental.pallas import tpu as pltpu
from jax.experimental.pallas import tpu_sc as plsc

F32 = jnp.float32
BF16 = jnp.bfloat16
_INTERPRET = False


def _pc(*a, **k):
    return pl.pallas_call(*a, interpret=_INTERPRET, **k)


def _ln(v, g, b):
    mu = jnp.mean(v, axis=-1, keepdims=True)
    var = jnp.mean((v - mu) * (v - mu), axis=-1, keepdims=True)
    return (v - mu) * jax.lax.rsqrt(var + 1e-5) * g + b


def _dot(a, b, prec=None):
    return jax.lax.dot_general(a, b, (((1,), (0,)), ((), ())),
                               preferred_element_type=F32, precision=prec)


_HI = jax.lax.Precision.HIGHEST


def _emb_body(x_ref, w_ref, b_ref, h_ref):
    h_ref[...] = _dot(x_ref[...], w_ref[...], _HI) + b_ref[...]


def _emb(x, w, b):
    n = x.shape[0]
    return _pc(
        _emb_body,
        out_shape=jax.ShapeDtypeStruct((n, w.shape[1]), F32),
    )(x, w, b.reshape(1, -1))


def _tables_body(h_ref, wa_ref, wb_ref, b1_ref, pa_ref, pb_ref):
    h = h_ref[...]
    pa_ref[...] = _dot(h, wa_ref[...], _HI) + b1_ref[...]
    pb_ref[...] = _dot(h, wb_ref[...], _HI)


def _tables(h, w1, b1, bn):
    n, d = h.shape
    wa = w1[:d]
    wb = w1[d:2 * d]
    grid = n // bn
    return _pc(
        _tables_body,
        grid=(grid,),
        in_specs=[
            pl.BlockSpec((bn, d), lambda i: (i, 0)),
            pl.BlockSpec((d, d), lambda i: (0, 0)),
            pl.BlockSpec((d, d), lambda i: (0, 0)),
            pl.BlockSpec((1, d), lambda i: (0, 0)),
        ],
        out_specs=[
            pl.BlockSpec((bn, d), lambda i: (i, 0)),
            pl.BlockSpec((bn, d), lambda i: (i, 0)),
        ],
        out_shape=[
            jax.ShapeDtypeStruct((n, d), F32),
            jax.ShapeDtypeStruct((n, d), F32),
        ],
    )(h, wa, wb, b1.reshape(1, -1))


def _edge_body(ga_ref, gb_ref, gpd_ref, gps_ref, w1d_ref,
               g1_ref, be1_ref, w2_ref, b2_ref, g2_ref, be2_ref,
               pw1_ref, pb1_ref, pg_ref, pbe_ref, pw2_ref, pb2_ref,
               m_ref, aux_ref):
    pd = gpd_ref[...][:, :3] - gps_ref[...][:, :3]
    d2 = jnp.sum(pd * pd, axis=-1, keepdims=True)
    dist = jnp.sqrt(d2 + 1e-12)
    m1 = ga_ref[...] + gb_ref[...] + dist * w1d_ref[...]
    m1 = jax.nn.relu(_ln(m1, g1_ref[...], be1_ref[...]))
    m = _dot(m1, w2_ref[...]) + b2_ref[...]
    m = jax.nn.relu(_ln(m, g2_ref[...], be2_ref[...]))
    w = _dot(m, pw1_ref[...]) + pb1_ref[...]
    w = jax.nn.relu(_ln(w, pg_ref[...], pbe_ref[...]))
    s = jnp.sum(w * pw2_ref[...], axis=-1, keepdims=True) + pb2_ref[0, 0]
    pdw = pd * s
    m_ref[...] = m
    be = pd.shape[0]
    aux = jnp.concatenate(
        [pdw, jnp.ones((be, 1), F32), jnp.zeros((be, 12), F32)], axis=-1)
    aux_ref[...] = aux


def _edge_mlp(ga, gb, gpd, gps, p, l, be):
    e, d = ga.shape
    grid = e // be
    row = lambda a: a.reshape(1, -1)
    w1d = p["msg_W1"][l][2 * d].reshape(1, d)
    return _pc(
        _edge_body,
        grid=(grid,),
        in_specs=[
            pl.BlockSpec((be, d), lambda i: (i, 0)),
            pl.BlockSpec((be, d), lambda i: (i, 0)),
            pl.BlockSpec((be, 16), lambda i: (i, 0)),
            pl.BlockSpec((be, 16), lambda i: (i, 0)),
            pl.BlockSpec((1, d), lambda i: (0, 0)),
            pl.BlockSpec((1, d), lambda i: (0, 0)),
            pl.BlockSpec((1, d), lambda i: (0, 0)),
            pl.BlockSpec((d, d), lambda i: (0, 0)),
            pl.BlockSpec((1, d), lambda i: (0, 0)),
            pl.BlockSpec((1, d), lambda i: (0, 0)),
            pl.BlockSpec((1, d), lambda i: (0, 0)),
            pl.BlockSpec((d, d), lambda i: (0, 0)),
            pl.BlockSpec((1, d), lambda i: (0, 0)),
            pl.BlockSpec((1, d), lambda i: (0, 0)),
            pl.BlockSpec((1, d), lambda i: (0, 0)),
            pl.BlockSpec((1, d), lambda i: (0, 0)),
            pl.BlockSpec((1, 1), lambda i: (0, 0)),
        ],
        out_specs=[
            pl.BlockSpec((be, d), lambda i: (i, 0)),
            pl.BlockSpec((be, 16), lambda i: (i, 0)),
        ],
        out_shape=[
            jax.ShapeDtypeStruct((e, d), F32),
            jax.ShapeDtypeStruct((e, 16), F32),
        ],
    )(ga, gb, gpd, gps, w1d,
      row(p["msg_g1"][l]), row(p["msg_be1"][l]),
      p["msg_W2"][l], row(p["msg_b2"][l]),
      row(p["msg_g2"][l]), row(p["msg_be2"][l]),
      p["pos_W1"][l], row(p["pos_b1"][l]),
      row(p["pos_g"][l]), row(p["pos_be"][l]),
      p["pos_W2"][l].reshape(1, d), p["pos_b2"][l].reshape(1, 1))


def _node_body(h_ref, posp_ref, pm0_ref, paux0_ref, pm1_ref, paux1_ref,
               u1a_ref, u1b_ref, ub1_ref, ug1_ref, ube1_ref,
               u2_ref, ub2_ref, ug2_ref, ube2_ref,
               hn_ref, posn_ref):
    h = h_ref[...]
    msg = pm0_ref[0] + pm0_ref[1] + pm1_ref[0] + pm1_ref[1]
    aux = paux0_ref[0] + paux0_ref[1] + paux1_ref[0] + paux1_ref[1]
    cnt = jnp.maximum(aux[:, 3:4], 1.0)
    pos_aggr = aux[:, :3] / cnt
    u = _dot(h, u1a_ref[...], _HI) + _dot(msg, u1b_ref[...], _HI) + ub1_ref[...]
    u = jax.nn.relu(_ln(u, ug1_ref[...], ube1_ref[...]))
    u = _dot(u, u2_ref[...], _HI) + ub2_ref[...]
    u = jax.nn.relu(_ln(u, ug2_ref[...], ube2_ref[...]))
    hn_ref[...] = h + u
    be = h.shape[0]
    posn_ref[...] = posp_ref[...] + jnp.concatenate(
        [pos_aggr, jnp.zeros((be, 13), F32)], axis=-1)


def _node_update(h, posp, pm0, paux0, pm1, paux1, p, l, bn):
    n, d = h.shape
    grid = n // bn
    row = lambda a: a.reshape(1, -1)
    u1 = p["upd_W1"][l]
    return _pc(
        _node_body,
        grid=(grid,),
        in_specs=[
            pl.BlockSpec((bn, d), lambda i: (i, 0)),
            pl.BlockSpec((bn, 16), lambda i: (i, 0)),
            pl.BlockSpec((2, bn, d), lambda i: (0, i, 0)),
            pl.BlockSpec((2, bn, 16), lambda i: (0, i, 0)),
            pl.BlockSpec((2, bn, d), lambda i: (0, i, 0)),
            pl.BlockSpec((2, bn, 16), lambda i: (0, i, 0)),
            pl.BlockSpec((d, d), lambda i: (0, 0)),
            pl.BlockSpec((d, d), lambda i: (0, 0)),
            pl.BlockSpec((1, d), lambda i: (0, 0)),
            pl.BlockSpec((1, d), lambda i: (0, 0)),
            pl.BlockSpec((1, d), lambda i: (0, 0)),
            pl.BlockSpec((d, d), lambda i: (0, 0)),
            pl.BlockSpec((1, d), lambda i: (0, 0)),
            pl.BlockSpec((1, d), lambda i: (0, 0)),
            pl.BlockSpec((1, d), lambda i: (0, 0)),
        ],
        out_specs=[
            pl.BlockSpec((bn, d), lambda i: (i, 0)),
            pl.BlockSpec((bn, 16), lambda i: (i, 0)),
        ],
        out_shape=[
            jax.ShapeDtypeStruct((n, d), F32),
            jax.ShapeDtypeStruct((n, 16), F32),
        ],
    )(h, posp, pm0, paux0, pm1, paux1,
      u1[:d], u1[d:2 * d], row(p["upd_b1"][l]),
      row(p["upd_g1"][l]), row(p["upd_be1"][l]),
      p["upd_W2"][l], row(p["upd_b2"][l]),
      row(p["upd_g2"][l]), row(p["upd_be2"][l]))


def _pred_body(h_ref, b_ref, w1_ref, b1_ref, w2_ref, b2_ref, o_ref):
    n, _ = h_ref.shape
    ng = o_ref.shape[0]
    seg = jax.lax.broadcasted_iota(jnp.int32, (n, ng), 1)
    onehot = jnp.where(b_ref[...] == seg, 1.0, 0.0).astype(F32)
    g = jax.lax.dot_general(onehot, h_ref[...], (((0,), (0,)), ((), ())),
                            preferred_element_type=F32, precision=_HI)
    u = jax.nn.relu(_dot(g, w1_ref[...], _HI) + b1_ref[...])
    o_ref[...] = _dot(u, w2_ref[...], _HI) + b2_ref[...]


def _pred(h, batch, p, ng):
    n, d = h.shape
    out_f = p["pred_W2"].shape[1]
    return _pc(
        _pred_body,
        out_shape=jax.ShapeDtypeStruct((ng, out_f), F32),
    )(h, batch.reshape(n, 1), p["pred_W1"], p["pred_b1"].reshape(1, -1),
      p["pred_W2"], p["pred_b2"].reshape(1, -1))


_NW = 32
_CHUNK = 80


def _sc_gather(pa, pb, posp, src2, dst2):
    n, d = pa.shape
    ch = src2.shape[1]
    e = src2.shape[0] * ch
    per_w = e // _NW
    nch = per_w // ch
    assert per_w * _NW == e and nch * ch == per_w and nch % 2 == 1
    mesh = plsc.VectorSubcoreMesh(core_axis_name="c", subcore_axis_name="s")

    @functools.partial(
        pl.kernel, mesh=mesh, interpret=_INTERPRET,
        compiler_params=pltpu.CompilerParams(use_tc_tiling_on_sc=False),
        out_type=[jax.ShapeDtypeStruct((e, d), F32),
                  jax.ShapeDtypeStruct((e, d), F32),
                  jax.ShapeDtypeStruct((e, 16), F32),
                  jax.ShapeDtypeStruct((e, 16), F32)],
        scratch_types=[pltpu.VMEM((nch, ch), jnp.int32),
                       pltpu.VMEM((nch, ch), jnp.int32)]
                      + [pltpu.VMEM((ch, d), F32),
                         pltpu.VMEM((ch, d), F32),
                         pltpu.VMEM((ch, 16), F32),
                         pltpu.VMEM((ch, 16), F32)] * 2
                      + [pltpu.SemaphoreType.DMA] * 4,
    )
    def k(pa_h, pb_h, posp_h, src_h, dst_h, ga_h, gb_h, gpd_h, gps_h,
          idxd_v, idxs_v, ra0, rb0, rpd0, rps0, ra1, rb1, rpd1, rps1,
          gsem0, gsem1, wsem0, wsem1):
        wid = lax.axis_index("s") * 2 + lax.axis_index("c")
        base = wid * per_w
        pltpu.sync_copy(dst_h.at[pl.ds(wid * nch, nch)], idxd_v)
        pltpu.sync_copy(src_h.at[pl.ds(wid * nch, nch)], idxs_v)
        bufs = ((ra0, rb0, rpd0, rps0, gsem0, wsem0),
                (ra1, rb1, rpd1, rps1, gsem1, wsem1))

        def g_descs(ci, bs):
            return (pltpu.make_async_copy(pa_h.at[idxd_v.at[ci]], bs[0], bs[4]),
                    pltpu.make_async_copy(pb_h.at[idxs_v.at[ci]], bs[1], bs[4]),
                    pltpu.make_async_copy(posp_h.at[idxd_v.at[ci]], bs[2], bs[4]),
                    pltpu.make_async_copy(posp_h.at[idxs_v.at[ci]], bs[3], bs[4]))

        def w_descs(ci, bs):
            off = base + ci * ch
            return (pltpu.make_async_copy(bs[0], ga_h.at[pl.ds(off, ch)], bs[5]),
                    pltpu.make_async_copy(bs[1], gb_h.at[pl.ds(off, ch)], bs[5]),
                    pltpu.make_async_copy(bs[2], gpd_h.at[pl.ds(off, ch)], bs[5]),
                    pltpu.make_async_copy(bs[3], gps_h.at[pl.ds(off, ch)], bs[5]))

        def start(ds):
            for dd in ds:
                dd.start()

        def wait(ds):
            for dd in ds:
                dd.wait()

        start(g_descs(0, bufs[0]))

        def body(i2, carry):
            i = i2 * 2
            a, b = bufs

            @pl.when(i > 0)
            def _():
                wait(w_descs(i - 1, b))

            start(g_descs(i + 1, b))
            wait(g_descs(i, a))
            start(w_descs(i, a))
            wait(g_descs(i + 1, b))
            wait(w_descs(i, a))

            @pl.when(i + 2 < nch)
            def _():
                start(g_descs(i + 2, a))

            start(w_descs(i + 1, b))
            return carry

        lax.fori_loop(0, (nch - 1) // 2, body, 0)
        last = nch - 1
        wait(w_descs(last - 1, bufs[1]))
        wait(g_descs(last, bufs[0]))
        start(w_descs(last, bufs[0]))
        wait(w_descs(last, bufs[0]))

    return k(pa, pb, posp, src2, dst2)


def _sc_scatter(m, aux, dst2, n):
    e, d = m.shape
    ch = dst2.shape[1]
    per_w = e // _NW
    nch = per_w // ch
    assert per_w * _NW == e and nch * ch == per_w and nch % 2 == 1
    rows_s = n // 16
    assert rows_s * 16 == n
    mesh = plsc.VectorSubcoreMesh(core_axis_name="c", subcore_axis_name="s")
    zm = jnp.zeros((rows_s, d), F32)
    za = jnp.zeros((rows_s, 16), F32)

    @functools.partial(
        pl.kernel, mesh=mesh, interpret=_INTERPRET,
        compiler_params=pltpu.CompilerParams(use_tc_tiling_on_sc=False),
        out_type=[jax.ShapeDtypeStruct((2, n, d), F32),
                  jax.ShapeDtypeStruct((2, n, 16), F32)],
        scratch_types=[pltpu.VMEM((nch, ch), jnp.int32)]
                      + [pltpu.VMEM((ch, d), F32),
                         pltpu.VMEM((ch, 16), F32)] * 2
                      + [pltpu.VMEM_SHARED((n, d), F32),
                         pltpu.VMEM_SHARED((n, 16), F32)]
                      + [pltpu.SemaphoreType.DMA] * 4,
    )
    def k(m_h, aux_h, dst_h, zm_h, za_h, pm_h, paux_h,
          idx_v, rm0, rx0, rm1, rx1, accm_s, acca_s,
          rsem0, rsem1, ssem0, ssem1):
        cid = lax.axis_index("c")
        sid = lax.axis_index("s")
        wid = sid * 2 + cid
        base = wid * per_w
        pltpu.sync_copy(zm_h, accm_s.at[pl.ds(sid * rows_s, rows_s)])
        pltpu.sync_copy(za_h, acca_s.at[pl.ds(sid * rows_s, rows_s)])
        pltpu.sync_copy(dst_h.at[pl.ds(wid * nch, nch)], idx_v)
        plsc.subcore_barrier()
        bufs = ((rm0, rx0, rsem0, ssem0), (rm1, rx1, rsem1, ssem1))

        def r_descs(ci, bs):
            off = base + ci * ch
            return (pltpu.make_async_copy(m_h.at[pl.ds(off, ch)], bs[0], bs[2]),
                    pltpu.make_async_copy(aux_h.at[pl.ds(off, ch)], bs[1], bs[2]))

        def s_descs(ci, bs):
            return (pltpu.make_async_copy(bs[0], accm_s.at[idx_v.at[ci]], bs[3]),
                    pltpu.make_async_copy(bs[1], acca_s.at[idx_v.at[ci]], bs[3]))

        def start(ds, add=False):
            for dd in ds:
                dd.start(add=add)

        def wait(ds):
            for dd in ds:
                dd.wait()

        start(r_descs(0, bufs[0]))

        def body(i2, carry):
            i = i2 * 2
            a, b = bufs

            @pl.when(i > 0)
            def _():
                wait(s_descs(i - 1, b))

            start(r_descs(i + 1, b))
            wait(r_descs(i, a))
            start(s_descs(i, a), add=True)
            wait(r_descs(i + 1, b))
            wait(s_descs(i, a))

            @pl.when(i + 2 < nch)
            def _():
                start(r_descs(i + 2, a))

            start(s_descs(i + 1, b), add=True)
            return carry

        lax.fori_loop(0, (nch - 1) // 2, body, 0)
        last = nch - 1
        wait(s_descs(last - 1, bufs[1]))
        wait(r_descs(last, bufs[0]))
        start(s_descs(last, bufs[0]), add=True)
        wait(s_descs(last, bufs[0]))
        plsc.subcore_barrier()
        pltpu.sync_copy(accm_s.at[pl.ds(sid * rows_s, rows_s)],
                        pm_h.at[cid, pl.ds(sid * rows_s, rows_s)])
        pltpu.sync_copy(acca_s.at[pl.ds(sid * rows_s, rows_s)],
                        paux_h.at[cid, pl.ds(sid * rows_s, rows_s)])

    return k(m, aux, dst2, zm, za)


def _gather(pa, pb, posp, src, dst):
    ga = jnp.take(pa, dst, axis=0)
    gb = jnp.take(pb, src, axis=0)
    gpd = jnp.take(posp, dst, axis=0)
    gps = jnp.take(posp, src, axis=0)
    return ga, gb, gpd, gps


def _scatter(m, aux, dst, n):
    pm = jax.ops.segment_sum(m, dst, num_segments=n)
    paux = jax.ops.segment_sum(aux, dst, num_segments=n)
    z = jnp.zeros_like(pm)
    za = jnp.zeros_like(paux)
    return jnp.stack([pm, z]), jnp.stack([paux, za])


def kernel(x, pos, params, edge_index, batch):
    p = params
    n = x.shape[0]
    e = edge_index.shape[1]
    depth = p["msg_W1"].shape[0]
    ch = 40
    nrows = e // ch
    src2 = edge_index[0].reshape(nrows, ch)
    dst2 = edge_index[1].reshape(nrows, ch)
    hrows = nrows // 2
    halves = ((src2[:hrows], dst2[:hrows]), (src2[hrows:], dst2[hrows:]))
    bn = 2000 if n % 2000 == 0 else n
    be = 2000 if (e // 2) % 2000 == 0 else e // 2

    posp = jnp.pad(pos, ((0, 0), (0, 13)))
    h = _emb(x, p["emb_W"], p["emb_b"])
    for l in range(depth):
        pa, pb = _tables(h, p["msg_W1"][l], p["msg_b1"][l], bn)
        g0 = _sc_gather(pa, pb, posp, *halves[0])
        g1 = _sc_gather(pa, pb, posp, *halves[1])
        m0, aux0 = _edge_mlp(*g0, p, l, be)
        m1, aux1 = _edge_mlp(*g1, p, l, be)
        pm0, paux0 = _sc_scatter(m0, aux0, halves[0][1], n)
        pm1, paux1 = _sc_scatter(m1, aux1, halves[1][1], n)
        h, posp = _node_update(h, posp, pm0, paux0, pm1, paux1, p, l, bn)
    return _pred(h, batch, p, 16)


# speedup vs baseline: 1.5372x; 1.0109x over previous
import functools

import jax
import jax.numpy as jnp
from jax import lax
from jax.experimental import pallas as pl
from jax.experimental.pallas import tpu as pltpu
from jax.experimental.pallas import tpu_sc as plsc

F32 = jnp.float32
BF16 = jnp.bfloat16
_INTERPRET = False


def _pc(*a, **k):
    return pl.pallas_call(*a, interpret=_INTERPRET, **k)


def _ln(v, g, b):
    mu = jnp.mean(v, axis=-1, keepdims=True)
    var = jnp.mean((v - mu) * (v - mu), axis=-1, keepdims=True)
    return (v - mu) * jax.lax.rsqrt(var + 1e-5) * g + b


def _dot(a, b, prec=None):
    return jax.lax.dot_general(a, b, (((1,), (0,)), ((), ())),
                               preferred_element_type=F32, precision=prec)


_HI = jax.lax.Precision.HIGHEST


def _emb_body(x_ref, w_ref, b_ref, h_ref):
    h_ref[...] = _dot(x_ref[...], w_ref[...]) + b_ref[...]


def _emb(x, w, b):
    n = x.shape[0]
    return _pc(
        _emb_body,
        out_shape=jax.ShapeDtypeStruct((n, w.shape[1]), F32),
    )(x, w, b.reshape(1, -1))


def _tables_body(h_ref, wa_ref, wb_ref, b1_ref, pa_ref, pb_ref):
    h = h_ref[...]
    pa_ref[...] = _dot(h, wa_ref[...]) + b1_ref[...]
    pb_ref[...] = _dot(h, wb_ref[...])


def _tables(h, w1, b1, bn):
    n, d = h.shape
    wa = w1[:d]
    wb = w1[d:2 * d]
    grid = n // bn
    return _pc(
        _tables_body,
        grid=(grid,),
        in_specs=[
            pl.BlockSpec((bn, d), lambda i: (i, 0)),
            pl.BlockSpec((d, d), lambda i: (0, 0)),
            pl.BlockSpec((d, d), lambda i: (0, 0)),
            pl.BlockSpec((1, d), lambda i: (0, 0)),
        ],
        out_specs=[
            pl.BlockSpec((bn, d), lambda i: (i, 0)),
            pl.BlockSpec((bn, d), lambda i: (i, 0)),
        ],
        out_shape=[
            jax.ShapeDtypeStruct((n, d), F32),
            jax.ShapeDtypeStruct((n, d), F32),
        ],
    )(h, wa, wb, b1.reshape(1, -1))


def _edge_body(ga_ref, gb_ref, gpd_ref, gps_ref, w1d_ref,
               g1_ref, be1_ref, w2_ref, b2_ref, g2_ref, be2_ref,
               pw1_ref, pb1_ref, pg_ref, pbe_ref, pw2_ref, pb2_ref,
               m_ref, aux_ref):
    pd = gpd_ref[...][:, :3] - gps_ref[...][:, :3]
    d2 = jnp.sum(pd * pd, axis=-1, keepdims=True)
    dist = jnp.sqrt(d2 + 1e-12)
    m1 = ga_ref[...] + gb_ref[...] + dist * w1d_ref[...]
    m1 = jax.nn.relu(_ln(m1, g1_ref[...], be1_ref[...]))
    m = _dot(m1, w2_ref[...]) + b2_ref[...]
    m = jax.nn.relu(_ln(m, g2_ref[...], be2_ref[...]))
    w = _dot(m, pw1_ref[...]) + pb1_ref[...]
    w = jax.nn.relu(_ln(w, pg_ref[...], pbe_ref[...]))
    s = jnp.sum(w * pw2_ref[...], axis=-1, keepdims=True) + pb2_ref[0, 0]
    pdw = pd * s
    m_ref[...] = m
    be = pd.shape[0]
    aux = jnp.concatenate(
        [pdw, jnp.ones((be, 1), F32), jnp.zeros((be, 12), F32)], axis=-1)
    aux_ref[...] = aux


def _edge_mlp(ga, gb, gpd, gps, p, l, be):
    e, d = ga.shape
    grid = e // be
    row = lambda a: a.reshape(1, -1)
    w1d = p["msg_W1"][l][2 * d].reshape(1, d)
    return _pc(
        _edge_body,
        grid=(grid,),
        in_specs=[
            pl.BlockSpec((be, d), lambda i: (i, 0)),
            pl.BlockSpec((be, d), lambda i: (i, 0)),
            pl.BlockSpec((be, 16), lambda i: (i, 0)),
            pl.BlockSpec((be, 16), lambda i: (i, 0)),
            pl.BlockSpec((1, d), lambda i: (0, 0)),
            pl.BlockSpec((1, d), lambda i: (0, 0)),
            pl.BlockSpec((1, d), lambda i: (0, 0)),
            pl.BlockSpec((d, d), lambda i: (0, 0)),
            pl.BlockSpec((1, d), lambda i: (0, 0)),
            pl.BlockSpec((1, d), lambda i: (0, 0)),
            pl.BlockSpec((1, d), lambda i: (0, 0)),
            pl.BlockSpec((d, d), lambda i: (0, 0)),
            pl.BlockSpec((1, d), lambda i: (0, 0)),
            pl.BlockSpec((1, d), lambda i: (0, 0)),
            pl.BlockSpec((1, d), lambda i: (0, 0)),
            pl.BlockSpec((1, d), lambda i: (0, 0)),
            pl.BlockSpec((1, 1), lambda i: (0, 0)),
        ],
        out_specs=[
            pl.BlockSpec((be, d), lambda i: (i, 0)),
            pl.BlockSpec((be, 16), lambda i: (i, 0)),
        ],
        out_shape=[
            jax.ShapeDtypeStruct((e, d), F32),
            jax.ShapeDtypeStruct((e, 16), F32),
        ],
    )(ga, gb, gpd, gps, w1d,
      row(p["msg_g1"][l]), row(p["msg_be1"][l]),
      p["msg_W2"][l], row(p["msg_b2"][l]),
      row(p["msg_g2"][l]), row(p["msg_be2"][l]),
      p["pos_W1"][l], row(p["pos_b1"][l]),
      row(p["pos_g"][l]), row(p["pos_be"][l]),
      p["pos_W2"][l].reshape(1, d), p["pos_b2"][l].reshape(1, 1))


def _node_body(h_ref, posp_ref, pm0_ref, paux0_ref, pm1_ref, paux1_ref,
               u1a_ref, u1b_ref, ub1_ref, ug1_ref, ube1_ref,
               u2_ref, ub2_ref, ug2_ref, ube2_ref,
               hn_ref, posn_ref):
    h = h_ref[...]
    msg = pm0_ref[0] + pm0_ref[1] + pm1_ref[0] + pm1_ref[1]
    aux = paux0_ref[0] + paux0_ref[1] + paux1_ref[0] + paux1_ref[1]
    cnt = jnp.maximum(aux[:, 3:4], 1.0)
    pos_aggr = aux[:, :3] / cnt
    u = _dot(h, u1a_ref[...]) + _dot(msg, u1b_ref[...]) + ub1_ref[...]
    u = jax.nn.relu(_ln(u, ug1_ref[...], ube1_ref[...]))
    u = _dot(u, u2_ref[...]) + ub2_ref[...]
    u = jax.nn.relu(_ln(u, ug2_ref[...], ube2_ref[...]))
    hn_ref[...] = h + u
    be = h.shape[0]
    posn_ref[...] = posp_ref[...] + jnp.concatenate(
        [pos_aggr, jnp.zeros((be, 13), F32)], axis=-1)


def _node_update(h, posp, pm0, paux0, pm1, paux1, p, l, bn):
    n, d = h.shape
    grid = n // bn
    row = lambda a: a.reshape(1, -1)
    u1 = p["upd_W1"][l]
    return _pc(
        _node_body,
        grid=(grid,),
        in_specs=[
            pl.BlockSpec((bn, d), lambda i: (i, 0)),
            pl.BlockSpec((bn, 16), lambda i: (i, 0)),
            pl.BlockSpec((2, bn, d), lambda i: (0, i, 0)),
            pl.BlockSpec((2, bn, 16), lambda i: (0, i, 0)),
            pl.BlockSpec((2, bn, d), lambda i: (0, i, 0)),
            pl.BlockSpec((2, bn, 16), lambda i: (0, i, 0)),
            pl.BlockSpec((d, d), lambda i: (0, 0)),
            pl.BlockSpec((d, d), lambda i: (0, 0)),
            pl.BlockSpec((1, d), lambda i: (0, 0)),
            pl.BlockSpec((1, d), lambda i: (0, 0)),
            pl.BlockSpec((1, d), lambda i: (0, 0)),
            pl.BlockSpec((d, d), lambda i: (0, 0)),
            pl.BlockSpec((1, d), lambda i: (0, 0)),
            pl.BlockSpec((1, d), lambda i: (0, 0)),
            pl.BlockSpec((1, d), lambda i: (0, 0)),
        ],
        out_specs=[
            pl.BlockSpec((bn, d), lambda i: (i, 0)),
            pl.BlockSpec((bn, 16), lambda i: (i, 0)),
        ],
        out_shape=[
            jax.ShapeDtypeStruct((n, d), F32),
            jax.ShapeDtypeStruct((n, 16), F32),
        ],
    )(h, posp, pm0, paux0, pm1, paux1,
      u1[:d], u1[d:2 * d], row(p["upd_b1"][l]),
      row(p["upd_g1"][l]), row(p["upd_be1"][l]),
      p["upd_W2"][l], row(p["upd_b2"][l]),
      row(p["upd_g2"][l]), row(p["upd_be2"][l]))


def _pred_body(h_ref, b_ref, w1_ref, b1_ref, w2_ref, b2_ref, o_ref):
    n, _ = h_ref.shape
    ng = o_ref.shape[0]
    seg = jax.lax.broadcasted_iota(jnp.int32, (n, ng), 1)
    onehot = jnp.where(b_ref[...] == seg, 1.0, 0.0).astype(F32)
    g = jax.lax.dot_general(onehot, h_ref[...], (((0,), (0,)), ((), ())),
                            preferred_element_type=F32, precision=_HI)
    u = jax.nn.relu(_dot(g, w1_ref[...]) + b1_ref[...])
    o_ref[...] = _dot(u, w2_ref[...]) + b2_ref[...]


def _pred(h, batch, p, ng):
    n, d = h.shape
    out_f = p["pred_W2"].shape[1]
    return _pc(
        _pred_body,
        out_shape=jax.ShapeDtypeStruct((ng, out_f), F32),
    )(h, batch.reshape(n, 1), p["pred_W1"], p["pred_b1"].reshape(1, -1),
      p["pred_W2"], p["pred_b2"].reshape(1, -1))


_NW = 32
_CHUNK = 80


def _sc_gather(pa, pb, posp, src2, dst2):
    n, d = pa.shape
    ch = src2.shape[1]
    e = src2.shape[0] * ch
    per_w = e // _NW
    nch = per_w // ch
    assert per_w * _NW == e and nch * ch == per_w and nch % 2 == 1
    mesh = plsc.VectorSubcoreMesh(core_axis_name="c", subcore_axis_name="s")

    @functools.partial(
        pl.kernel, mesh=mesh, interpret=_INTERPRET,
        compiler_params=pltpu.CompilerParams(use_tc_tiling_on_sc=False),
        out_type=[jax.ShapeDtypeStruct((e, d), F32),
                  jax.ShapeDtypeStruct((e, d), F32),
                  jax.ShapeDtypeStruct((e, 16), F32),
                  jax.ShapeDtypeStruct((e, 16), F32)],
        scratch_types=[pltpu.VMEM((nch, ch), jnp.int32),
                       pltpu.VMEM((nch, ch), jnp.int32)]
                      + [pltpu.VMEM((ch, d), F32),
                         pltpu.VMEM((ch, d), F32),
                         pltpu.VMEM((ch, 16), F32),
                         pltpu.VMEM((ch, 16), F32)] * 2
                      + [pltpu.SemaphoreType.DMA] * 4,
    )
    def k(pa_h, pb_h, posp_h, src_h, dst_h, ga_h, gb_h, gpd_h, gps_h,
          idxd_v, idxs_v, ra0, rb0, rpd0, rps0, ra1, rb1, rpd1, rps1,
          gsem0, gsem1, wsem0, wsem1):
        wid = lax.axis_index("s") * 2 + lax.axis_index("c")
        base = wid * per_w
        pltpu.sync_copy(dst_h.at[pl.ds(wid * nch, nch)], idxd_v)
        pltpu.sync_copy(src_h.at[pl.ds(wid * nch, nch)], idxs_v)
        bufs = ((ra0, rb0, rpd0, rps0, gsem0, wsem0),
                (ra1, rb1, rpd1, rps1, gsem1, wsem1))

        def g_descs(ci, bs):
            return (pltpu.make_async_copy(pa_h.at[idxd_v.at[ci]], bs[0], bs[4]),
                    pltpu.make_async_copy(pb_h.at[idxs_v.at[ci]], bs[1], bs[4]),
                    pltpu.make_async_copy(posp_h.at[idxd_v.at[ci]], bs[2], bs[4]),
                    pltpu.make_async_copy(posp_h.at[idxs_v.at[ci]], bs[3], bs[4]))

        def w_descs(ci, bs):
            off = base + ci * ch
            return (pltpu.make_async_copy(bs[0], ga_h.at[pl.ds(off, ch)], bs[5]),
                    pltpu.make_async_copy(bs[1], gb_h.at[pl.ds(off, ch)], bs[5]),
                    pltpu.make_async_copy(bs[2], gpd_h.at[pl.ds(off, ch)], bs[5]),
                    pltpu.make_async_copy(bs[3], gps_h.at[pl.ds(off, ch)], bs[5]))

        def start(ds):
            for dd in ds:
                dd.start()

        def wait(ds):
            for dd in ds:
                dd.wait()

        start(g_descs(0, bufs[0]))

        def body(i2, carry):
            i = i2 * 2
            a, b = bufs

            @pl.when(i > 0)
            def _():
                wait(w_descs(i - 1, b))

            start(g_descs(i + 1, b))
            wait(g_descs(i, a))
            start(w_descs(i, a))
            wait(g_descs(i + 1, b))
            wait(w_descs(i, a))

            @pl.when(i + 2 < nch)
            def _():
                start(g_descs(i + 2, a))

            start(w_descs(i + 1, b))
            return carry

        lax.fori_loop(0, (nch - 1) // 2, body, 0)
        last = nch - 1
        wait(w_descs(last - 1, bufs[1]))
        wait(g_descs(last, bufs[0]))
        start(w_descs(last, bufs[0]))
        wait(w_descs(last, bufs[0]))

    return k(pa, pb, posp, src2, dst2)


def _sc_scatter(m, aux, dst2, n):
    e, d = m.shape
    ch = dst2.shape[1]
    per_w = e // _NW
    nch = per_w // ch
    assert per_w * _NW == e and nch * ch == per_w and nch % 2 == 1
    rows_s = n // 16
    assert rows_s * 16 == n
    mesh = plsc.VectorSubcoreMesh(core_axis_name="c", subcore_axis_name="s")
    zm = jnp.zeros((rows_s, d), F32)
    za = jnp.zeros((rows_s, 16), F32)

    @functools.partial(
        pl.kernel, mesh=mesh, interpret=_INTERPRET,
        compiler_params=pltpu.CompilerParams(use_tc_tiling_on_sc=False),
        out_type=[jax.ShapeDtypeStruct((2, n, d), F32),
                  jax.ShapeDtypeStruct((2, n, 16), F32)],
        scratch_types=[pltpu.VMEM((nch, ch), jnp.int32)]
                      + [pltpu.VMEM((ch, d), F32),
                         pltpu.VMEM((ch, 16), F32)] * 2
                      + [pltpu.VMEM_SHARED((n, d), F32),
                         pltpu.VMEM_SHARED((n, 16), F32)]
                      + [pltpu.SemaphoreType.DMA] * 4,
    )
    def k(m_h, aux_h, dst_h, zm_h, za_h, pm_h, paux_h,
          idx_v, rm0, rx0, rm1, rx1, accm_s, acca_s,
          rsem0, rsem1, ssem0, ssem1):
        cid = lax.axis_index("c")
        sid = lax.axis_index("s")
        wid = sid * 2 + cid
        base = wid * per_w
        pltpu.sync_copy(zm_h, accm_s.at[pl.ds(sid * rows_s, rows_s)])
        pltpu.sync_copy(za_h, acca_s.at[pl.ds(sid * rows_s, rows_s)])
        pltpu.sync_copy(dst_h.at[pl.ds(wid * nch, nch)], idx_v)
        plsc.subcore_barrier()
        bufs = ((rm0, rx0, rsem0, ssem0), (rm1, rx1, rsem1, ssem1))

        def r_descs(ci, bs):
            off = base + ci * ch
            return (pltpu.make_async_copy(m_h.at[pl.ds(off, ch)], bs[0], bs[2]),
                    pltpu.make_async_copy(aux_h.at[pl.ds(off, ch)], bs[1], bs[2]))

        def s_descs(ci, bs):
            return (pltpu.make_async_copy(bs[0], accm_s.at[idx_v.at[ci]], bs[3]),
                    pltpu.make_async_copy(bs[1], acca_s.at[idx_v.at[ci]], bs[3]))

        def start(ds, add=False):
            for dd in ds:
                dd.start(add=add)

        def wait(ds):
            for dd in ds:
                dd.wait()

        start(r_descs(0, bufs[0]))

        def body(i2, carry):
            i = i2 * 2
            a, b = bufs

            @pl.when(i > 0)
            def _():
                wait(s_descs(i - 1, b))

            start(r_descs(i + 1, b))
            wait(r_descs(i, a))
            start(s_descs(i, a), add=True)
            wait(r_descs(i + 1, b))
            wait(s_descs(i, a))

            @pl.when(i + 2 < nch)
            def _():
                start(r_descs(i + 2, a))

            start(s_descs(i + 1, b), add=True)
            return carry

        lax.fori_loop(0, (nch - 1) // 2, body, 0)
        last = nch - 1
        wait(s_descs(last - 1, bufs[1]))
        wait(r_descs(last, bufs[0]))
        start(s_descs(last, bufs[0]), add=True)
        wait(s_descs(last, bufs[0]))
        plsc.subcore_barrier()
        pltpu.sync_copy(accm_s.at[pl.ds(sid * rows_s, rows_s)],
                        pm_h.at[cid, pl.ds(sid * rows_s, rows_s)])
        pltpu.sync_copy(acca_s.at[pl.ds(sid * rows_s, rows_s)],
                        paux_h.at[cid, pl.ds(sid * rows_s, rows_s)])

    return k(m, aux, dst2, zm, za)


def _gather(pa, pb, posp, src, dst):
    ga = jnp.take(pa, dst, axis=0)
    gb = jnp.take(pb, src, axis=0)
    gpd = jnp.take(posp, dst, axis=0)
    gps = jnp.take(posp, src, axis=0)
    return ga, gb, gpd, gps


def _scatter(m, aux, dst, n):
    pm = jax.ops.segment_sum(m, dst, num_segments=n)
    paux = jax.ops.segment_sum(aux, dst, num_segments=n)
    z = jnp.zeros_like(pm)
    za = jnp.zeros_like(paux)
    return jnp.stack([pm, z]), jnp.stack([paux, za])


def kernel(x, pos, params, edge_index, batch):
    p = params
    n = x.shape[0]
    e = edge_index.shape[1]
    depth = p["msg_W1"].shape[0]
    ch = 40
    nrows = e // ch
    src2 = edge_index[0].reshape(nrows, ch)
    dst2 = edge_index[1].reshape(nrows, ch)
    hrows = nrows // 2
    halves = ((src2[:hrows], dst2[:hrows]), (src2[hrows:], dst2[hrows:]))
    bn = 2000 if n % 2000 == 0 else n
    be = 2000 if (e // 2) % 2000 == 0 else e // 2

    posp = jnp.pad(pos, ((0, 0), (0, 13)))
    h = _emb(x, p["emb_W"], p["emb_b"])
    for l in range(depth):
        pa, pb = _tables(h, p["msg_W1"][l], p["msg_b1"][l], bn)
        g0 = _sc_gather(pa, pb, posp, *halves[0])
        g1 = _sc_gather(pa, pb, posp, *halves[1])
        m0, aux0 = _edge_mlp(*g0, p, l, be)
        m1, aux1 = _edge_mlp(*g1, p, l, be)
        pm0, paux0 = _sc_scatter(m0, aux0, halves[0][1], n)
        pm1, paux1 = _sc_scatter(m1, aux1, halves[1][1], n)
        h, posp = _node_update(h, posp, pm0, paux0, pm1, paux1, p, l, bn)
    return _pred(h, batch, p, 16)


# speedup vs baseline: 1.5905x; 1.0347x over previous
import functools

import jax
import jax.numpy as jnp
from jax import lax
from jax.experimental import pallas as pl
from jax.experimental.pallas import tpu as pltpu
from jax.experimental.pallas import tpu_sc as plsc

F32 = jnp.float32
BF16 = jnp.bfloat16
_INTERPRET = False


def _pc(*a, **k):
    return pl.pallas_call(*a, interpret=_INTERPRET, **k)


def _ln(v, g, b):
    mu = jnp.mean(v, axis=-1, keepdims=True)
    var = jnp.mean((v - mu) * (v - mu), axis=-1, keepdims=True)
    return (v - mu) * jax.lax.rsqrt(var + 1e-5) * g + b


def _dot(a, b, prec=None):
    return jax.lax.dot_general(a, b, (((1,), (0,)), ((), ())),
                               preferred_element_type=F32, precision=prec)


_HI = jax.lax.Precision.HIGHEST


def _emb_body(x_ref, w_ref, b_ref, h_ref):
    h_ref[...] = _dot(x_ref[...], w_ref[...]) + b_ref[...]


def _emb(x, w, b):
    n = x.shape[0]
    return _pc(
        _emb_body,
        out_shape=jax.ShapeDtypeStruct((n, w.shape[1]), F32),
    )(x, w, b.reshape(1, -1))


def _tables_body(h_ref, wa_ref, wb_ref, b1_ref, pa_ref, pb_ref):
    h = h_ref[...]
    pa_ref[...] = _dot(h, wa_ref[...]) + b1_ref[...]
    pb_ref[...] = _dot(h, wb_ref[...])


def _tables(h, w1, b1, bn):
    n, d = h.shape
    wa = w1[:d]
    wb = w1[d:2 * d]
    grid = n // bn
    return _pc(
        _tables_body,
        grid=(grid,),
        in_specs=[
            pl.BlockSpec((bn, d), lambda i: (i, 0)),
            pl.BlockSpec((d, d), lambda i: (0, 0)),
            pl.BlockSpec((d, d), lambda i: (0, 0)),
            pl.BlockSpec((1, d), lambda i: (0, 0)),
        ],
        out_specs=[
            pl.BlockSpec((bn, d), lambda i: (i, 0)),
            pl.BlockSpec((bn, d), lambda i: (i, 0)),
        ],
        out_shape=[
            jax.ShapeDtypeStruct((n, d), F32),
            jax.ShapeDtypeStruct((n, d), F32),
        ],
    )(h, wa, wb, b1.reshape(1, -1))


def _edge_body(gm_ref, gpd_ref, w1d_ref,
               g1_ref, be1_ref, w2_ref, b2_ref, g2_ref, be2_ref,
               pw1_ref, pb1_ref, pg_ref, pbe_ref, pw2_ref, pb2_ref,
               m_ref, aux_ref):
    pd = gpd_ref[...][:, :3]
    d2 = jnp.sum(pd * pd, axis=-1, keepdims=True)
    dist = jnp.sqrt(d2 + 1e-12)
    m1 = gm_ref[...] + dist * w1d_ref[...]
    m1 = jax.nn.relu(_ln(m1, g1_ref[...], be1_ref[...]))
    m = _dot(m1, w2_ref[...]) + b2_ref[...]
    m = jax.nn.relu(_ln(m, g2_ref[...], be2_ref[...]))
    w = _dot(m, pw1_ref[...]) + pb1_ref[...]
    w = jax.nn.relu(_ln(w, pg_ref[...], pbe_ref[...]))
    s = jnp.sum(w * pw2_ref[...], axis=-1, keepdims=True) + pb2_ref[0, 0]
    pdw = pd * s
    m_ref[...] = m
    be = pd.shape[0]
    aux = jnp.concatenate(
        [pdw, jnp.ones((be, 1), F32), jnp.zeros((be, 12), F32)], axis=-1)
    aux_ref[...] = aux


def _edge_mlp(gm, gpd, p, l, be):
    e, d = gm.shape
    grid = e // be
    row = lambda a: a.reshape(1, -1)
    w1d = p["msg_W1"][l][2 * d].reshape(1, d)
    return _pc(
        _edge_body,
        grid=(grid,),
        in_specs=[
            pl.BlockSpec((be, d), lambda i: (i, 0)),
            pl.BlockSpec((be, 16), lambda i: (i, 0)),
            pl.BlockSpec((1, d), lambda i: (0, 0)),
            pl.BlockSpec((1, d), lambda i: (0, 0)),
            pl.BlockSpec((1, d), lambda i: (0, 0)),
            pl.BlockSpec((d, d), lambda i: (0, 0)),
            pl.BlockSpec((1, d), lambda i: (0, 0)),
            pl.BlockSpec((1, d), lambda i: (0, 0)),
            pl.BlockSpec((1, d), lambda i: (0, 0)),
            pl.BlockSpec((d, d), lambda i: (0, 0)),
            pl.BlockSpec((1, d), lambda i: (0, 0)),
            pl.BlockSpec((1, d), lambda i: (0, 0)),
            pl.BlockSpec((1, d), lambda i: (0, 0)),
            pl.BlockSpec((1, d), lambda i: (0, 0)),
            pl.BlockSpec((1, 1), lambda i: (0, 0)),
        ],
        out_specs=[
            pl.BlockSpec((be, d), lambda i: (i, 0)),
            pl.BlockSpec((be, 16), lambda i: (i, 0)),
        ],
        out_shape=[
            jax.ShapeDtypeStruct((e, d), F32),
            jax.ShapeDtypeStruct((e, 16), F32),
        ],
    )(gm, gpd, w1d,
      row(p["msg_g1"][l]), row(p["msg_be1"][l]),
      p["msg_W2"][l], row(p["msg_b2"][l]),
      row(p["msg_g2"][l]), row(p["msg_be2"][l]),
      p["pos_W1"][l], row(p["pos_b1"][l]),
      row(p["pos_g"][l]), row(p["pos_be"][l]),
      p["pos_W2"][l].reshape(1, d), p["pos_b2"][l].reshape(1, 1))


def _node_body(h_ref, posp_ref, pm0_ref, paux0_ref, pm1_ref, paux1_ref,
               u1a_ref, u1b_ref, ub1_ref, ug1_ref, ube1_ref,
               u2_ref, ub2_ref, ug2_ref, ube2_ref,
               hn_ref, posn_ref):
    h = h_ref[...]
    msg = pm0_ref[0] + pm0_ref[1] + pm1_ref[0] + pm1_ref[1]
    aux = paux0_ref[0] + paux0_ref[1] + paux1_ref[0] + paux1_ref[1]
    cnt = jnp.maximum(aux[:, 3:4], 1.0)
    pos_aggr = aux[:, :3] / cnt
    u = _dot(h, u1a_ref[...]) + _dot(msg, u1b_ref[...]) + ub1_ref[...]
    u = jax.nn.relu(_ln(u, ug1_ref[...], ube1_ref[...]))
    u = _dot(u, u2_ref[...]) + ub2_ref[...]
    u = jax.nn.relu(_ln(u, ug2_ref[...], ube2_ref[...]))
    hn_ref[...] = h + u
    be = h.shape[0]
    posn_ref[...] = posp_ref[...] + jnp.concatenate(
        [pos_aggr, jnp.zeros((be, 13), F32)], axis=-1)


def _node_update(h, posp, pm0, paux0, pm1, paux1, p, l, bn):
    n, d = h.shape
    grid = n // bn
    row = lambda a: a.reshape(1, -1)
    u1 = p["upd_W1"][l]
    return _pc(
        _node_body,
        grid=(grid,),
        in_specs=[
            pl.BlockSpec((bn, d), lambda i: (i, 0)),
            pl.BlockSpec((bn, 16), lambda i: (i, 0)),
            pl.BlockSpec((2, bn, d), lambda i: (0, i, 0)),
            pl.BlockSpec((2, bn, 16), lambda i: (0, i, 0)),
            pl.BlockSpec((2, bn, d), lambda i: (0, i, 0)),
            pl.BlockSpec((2, bn, 16), lambda i: (0, i, 0)),
            pl.BlockSpec((d, d), lambda i: (0, 0)),
            pl.BlockSpec((d, d), lambda i: (0, 0)),
            pl.BlockSpec((1, d), lambda i: (0, 0)),
            pl.BlockSpec((1, d), lambda i: (0, 0)),
            pl.BlockSpec((1, d), lambda i: (0, 0)),
            pl.BlockSpec((d, d), lambda i: (0, 0)),
            pl.BlockSpec((1, d), lambda i: (0, 0)),
            pl.BlockSpec((1, d), lambda i: (0, 0)),
            pl.BlockSpec((1, d), lambda i: (0, 0)),
        ],
        out_specs=[
            pl.BlockSpec((bn, d), lambda i: (i, 0)),
            pl.BlockSpec((bn, 16), lambda i: (i, 0)),
        ],
        out_shape=[
            jax.ShapeDtypeStruct((n, d), F32),
            jax.ShapeDtypeStruct((n, 16), F32),
        ],
    )(h, posp, pm0, paux0, pm1, paux1,
      u1[:d], u1[d:2 * d], row(p["upd_b1"][l]),
      row(p["upd_g1"][l]), row(p["upd_be1"][l]),
      p["upd_W2"][l], row(p["upd_b2"][l]),
      row(p["upd_g2"][l]), row(p["upd_be2"][l]))


def _pred_body(h_ref, b_ref, w1_ref, b1_ref, w2_ref, b2_ref, o_ref):
    n, _ = h_ref.shape
    ng = o_ref.shape[0]
    seg = jax.lax.broadcasted_iota(jnp.int32, (n, ng), 1)
    onehot = jnp.where(b_ref[...] == seg, 1.0, 0.0).astype(F32)
    g = jax.lax.dot_general(onehot, h_ref[...], (((0,), (0,)), ((), ())),
                            preferred_element_type=F32, precision=_HI)
    u = jax.nn.relu(_dot(g, w1_ref[...]) + b1_ref[...])
    o_ref[...] = _dot(u, w2_ref[...]) + b2_ref[...]


def _pred(h, batch, p, ng):
    n, d = h.shape
    out_f = p["pred_W2"].shape[1]
    return _pc(
        _pred_body,
        out_shape=jax.ShapeDtypeStruct((ng, out_f), F32),
    )(h, batch.reshape(n, 1), p["pred_W1"], p["pred_b1"].reshape(1, -1),
      p["pred_W2"], p["pred_b2"].reshape(1, -1))


_NW = 32
_CHUNK = 80


def _sc_gather(pa, pb, posp, src2, dst2):
    n, d = pa.shape
    ch = src2.shape[1]
    e = src2.shape[0] * ch
    per_w = e // _NW
    nch = per_w // ch
    assert per_w * _NW == e and nch * ch == per_w and nch % 2 == 1
    mesh = plsc.VectorSubcoreMesh(core_axis_name="c", subcore_axis_name="s")

    @functools.partial(
        pl.kernel, mesh=mesh, interpret=_INTERPRET,
        compiler_params=pltpu.CompilerParams(use_tc_tiling_on_sc=False),
        out_type=[jax.ShapeDtypeStruct((e, d), F32),
                  jax.ShapeDtypeStruct((e, 16), F32)],
        scratch_types=[pltpu.VMEM((nch, ch), jnp.int32),
                       pltpu.VMEM((nch, ch), jnp.int32)]
                      + [pltpu.VMEM((ch, d), F32),
                         pltpu.VMEM((ch, d), F32),
                         pltpu.VMEM((ch, 16), F32),
                         pltpu.VMEM((ch, 16), F32)] * 2
                      + [pltpu.SemaphoreType.DMA] * 4,
    )
    def k(pa_h, pb_h, posp_h, src_h, dst_h, gm_h, gpd_h,
          idxd_v, idxs_v, ra0, rb0, rpd0, rps0, ra1, rb1, rpd1, rps1,
          gsem0, gsem1, wsem0, wsem1):
        wid = lax.axis_index("s") * 2 + lax.axis_index("c")
        base = wid * per_w
        pltpu.sync_copy(dst_h.at[pl.ds(wid * nch, nch)], idxd_v)
        pltpu.sync_copy(src_h.at[pl.ds(wid * nch, nch)], idxs_v)
        bufs = ((ra0, rb0, rpd0, rps0, gsem0, wsem0),
                (ra1, rb1, rpd1, rps1, gsem1, wsem1))

        def g_descs(ci, bs):
            return (pltpu.make_async_copy(pa_h.at[idxd_v.at[ci]], bs[0], bs[4]),
                    pltpu.make_async_copy(pb_h.at[idxs_v.at[ci]], bs[1], bs[4]),
                    pltpu.make_async_copy(posp_h.at[idxd_v.at[ci]], bs[2], bs[4]),
                    pltpu.make_async_copy(posp_h.at[idxs_v.at[ci]], bs[3], bs[4]))

        def w_descs(ci, bs):
            off = base + ci * ch
            return (pltpu.make_async_copy(bs[0], gm_h.at[pl.ds(off, ch)], bs[5]),
                    pltpu.make_async_copy(bs[2], gpd_h.at[pl.ds(off, ch)], bs[5]))

        def fuse(bs):
            def row_body(r, carry):
                for j in range(d // 16):
                    sl = pl.ds(j * 16, 16)
                    bs[0][r, sl] = bs[0][r, sl] + bs[1][r, sl]
                sp = pl.ds(0, 16)
                bs[2][r, sp] = bs[2][r, sp] - bs[3][r, sp]
                return carry

            lax.fori_loop(0, ch, row_body, 0)

        def start(ds):
            for dd in ds:
                dd.start()

        def wait(ds):
            for dd in ds:
                dd.wait()

        start(g_descs(0, bufs[0]))

        def body(i2, carry):
            i = i2 * 2
            a, b = bufs

            @pl.when(i > 0)
            def _():
                wait(w_descs(i - 1, b))

            start(g_descs(i + 1, b))
            wait(g_descs(i, a))
            fuse(a)
            start(w_descs(i, a))
            wait(g_descs(i + 1, b))
            fuse(b)
            wait(w_descs(i, a))

            @pl.when(i + 2 < nch)
            def _():
                start(g_descs(i + 2, a))

            start(w_descs(i + 1, b))
            return carry

        lax.fori_loop(0, (nch - 1) // 2, body, 0)
        last = nch - 1
        wait(w_descs(last - 1, bufs[1]))
        wait(g_descs(last, bufs[0]))
        fuse(bufs[0])
        start(w_descs(last, bufs[0]))
        wait(w_descs(last, bufs[0]))

    return k(pa, pb, posp, src2, dst2)


def _sc_scatter(m, aux, dst2, n):
    e, d = m.shape
    ch = dst2.shape[1]
    per_w = e // _NW
    nch = per_w // ch
    assert per_w * _NW == e and nch * ch == per_w and nch % 2 == 1
    rows_s = n // 16
    assert rows_s * 16 == n
    mesh = plsc.VectorSubcoreMesh(core_axis_name="c", subcore_axis_name="s")
    zm = jnp.zeros((rows_s, d), F32)
    za = jnp.zeros((rows_s, 16), F32)

    @functools.partial(
        pl.kernel, mesh=mesh, interpret=_INTERPRET,
        compiler_params=pltpu.CompilerParams(use_tc_tiling_on_sc=False),
        out_type=[jax.ShapeDtypeStruct((2, n, d), F32),
                  jax.ShapeDtypeStruct((2, n, 16), F32)],
        scratch_types=[pltpu.VMEM((nch, ch), jnp.int32)]
                      + [pltpu.VMEM((ch, d), F32),
                         pltpu.VMEM((ch, 16), F32)] * 2
                      + [pltpu.VMEM_SHARED((n, d), F32),
                         pltpu.VMEM_SHARED((n, 16), F32)]
                      + [pltpu.SemaphoreType.DMA] * 4,
    )
    def k(m_h, aux_h, dst_h, zm_h, za_h, pm_h, paux_h,
          idx_v, rm0, rx0, rm1, rx1, accm_s, acca_s,
          rsem0, rsem1, ssem0, ssem1):
        cid = lax.axis_index("c")
        sid = lax.axis_index("s")
        wid = sid * 2 + cid
        base = wid * per_w
        pltpu.sync_copy(zm_h, accm_s.at[pl.ds(sid * rows_s, rows_s)])
        pltpu.sync_copy(za_h, acca_s.at[pl.ds(sid * rows_s, rows_s)])
        pltpu.sync_copy(dst_h.at[pl.ds(wid * nch, nch)], idx_v)
        plsc.subcore_barrier()
        bufs = ((rm0, rx0, rsem0, ssem0), (rm1, rx1, rsem1, ssem1))

        def r_descs(ci, bs):
            off = base + ci * ch
            return (pltpu.make_async_copy(m_h.at[pl.ds(off, ch)], bs[0], bs[2]),
                    pltpu.make_async_copy(aux_h.at[pl.ds(off, ch)], bs[1], bs[2]))

        def s_descs(ci, bs):
            return (pltpu.make_async_copy(bs[0], accm_s.at[idx_v.at[ci]], bs[3]),
                    pltpu.make_async_copy(bs[1], acca_s.at[idx_v.at[ci]], bs[3]))

        def start(ds, add=False):
            for dd in ds:
                dd.start(add=add)

        def wait(ds):
            for dd in ds:
                dd.wait()

        start(r_descs(0, bufs[0]))

        def body(i2, carry):
            i = i2 * 2
            a, b = bufs

            @pl.when(i > 0)
            def _():
                wait(s_descs(i - 1, b))

            start(r_descs(i + 1, b))
            wait(r_descs(i, a))
            start(s_descs(i, a), add=True)
            wait(r_descs(i + 1, b))
            wait(s_descs(i, a))

            @pl.when(i + 2 < nch)
            def _():
                start(r_descs(i + 2, a))

            start(s_descs(i + 1, b), add=True)
            return carry

        lax.fori_loop(0, (nch - 1) // 2, body, 0)
        last = nch - 1
        wait(s_descs(last - 1, bufs[1]))
        wait(r_descs(last, bufs[0]))
        start(s_descs(last, bufs[0]), add=True)
        wait(s_descs(last, bufs[0]))
        plsc.subcore_barrier()
        pltpu.sync_copy(accm_s.at[pl.ds(sid * rows_s, rows_s)],
                        pm_h.at[cid, pl.ds(sid * rows_s, rows_s)])
        pltpu.sync_copy(acca_s.at[pl.ds(sid * rows_s, rows_s)],
                        paux_h.at[cid, pl.ds(sid * rows_s, rows_s)])

    return k(m, aux, dst2, zm, za)


def _gather(pa, pb, posp, src, dst):
    ga = jnp.take(pa, dst, axis=0)
    gb = jnp.take(pb, src, axis=0)
    gpd = jnp.take(posp, dst, axis=0)
    gps = jnp.take(posp, src, axis=0)
    return ga, gb, gpd, gps


def _scatter(m, aux, dst, n):
    pm = jax.ops.segment_sum(m, dst, num_segments=n)
    paux = jax.ops.segment_sum(aux, dst, num_segments=n)
    z = jnp.zeros_like(pm)
    za = jnp.zeros_like(paux)
    return jnp.stack([pm, z]), jnp.stack([paux, za])


def kernel(x, pos, params, edge_index, batch):
    p = params
    n = x.shape[0]
    e = edge_index.shape[1]
    depth = p["msg_W1"].shape[0]
    ch = 40
    nrows = e // ch
    src2 = edge_index[0].reshape(nrows, ch)
    dst2 = edge_index[1].reshape(nrows, ch)
    hrows = nrows // 2
    halves = ((src2[:hrows], dst2[:hrows]), (src2[hrows:], dst2[hrows:]))
    bn = 2000 if n % 2000 == 0 else n
    be = 2000 if (e // 2) % 2000 == 0 else e // 2

    posp = jnp.pad(pos, ((0, 0), (0, 13)))
    h = _emb(x, p["emb_W"], p["emb_b"])
    for l in range(depth):
        pa, pb = _tables(h, p["msg_W1"][l], p["msg_b1"][l], bn)
        g0 = _sc_gather(pa, pb, posp, *halves[0])
        g1 = _sc_gather(pa, pb, posp, *halves[1])
        m0, aux0 = _edge_mlp(*g0, p, l, be)
        m1, aux1 = _edge_mlp(*g1, p, l, be)
        pm0, paux0 = _sc_scatter(m0, aux0, halves[0][1], n)
        pm1, paux1 = _sc_scatter(m1, aux1, halves[1][1], n)
        h, posp = _node_update(h, posp, pm0, paux0, pm1, paux1, p, l, bn)
    return _pred(h, batch, p, 16)
